# baseline (speedup 1.0000x reference)
import dataclasses
import functools

import jax
import jax.numpy as jnp
from jax import lax
from jax.experimental import pallas as pl
from jax.experimental.pallas import tpu as pltpu
from jax.experimental.pallas import tpu_sc as plsc

_NUM_STARS = 10
_NUM_POSTAL = 1000
_D = 128
_B = 16384

_NC = 2
_NS = 16
_L = 16
_NW = _NC * _NS
_ITEMS_PER_W = _B // _NW
_ROWS_PER_W = 2 * _ITEMS_PER_W
_CHUNK = 128
_N_CHUNKS = _ROWS_PER_W // _CHUNK
_NBUF = 4


_CP = pltpu.CompilerParams()
if "needs_layout_passes" in pltpu.CompilerParams.__dataclass_fields__:
    _CP = dataclasses.replace(_CP, needs_layout_passes=False)


@functools.partial(
    pl.kernel,
    mesh=plsc.VectorSubcoreMesh(core_axis_name="c", subcore_axis_name="s"),
    compiler_params=_CP,
    out_type=jax.ShapeDtypeStruct((_B, 2 * _D), jnp.float32),
    scratch_types=[
        pltpu.VMEM((_ITEMS_PER_W * 3,), jnp.int32),
        pltpu.VMEM((_ROWS_PER_W,), jnp.int32),
        pltpu.VMEM((_NBUF, _CHUNK, _D), jnp.float32),
        pltpu.VMEM_SHARED((_NUM_STARS + _NUM_POSTAL, _D), jnp.float32),
        pltpu.SemaphoreType.DMA,
        pltpu.SemaphoreType.DMA,
        pltpu.SemaphoreType.DMA,
        pltpu.SemaphoreType.DMA,
    ],
)
def _emb_lookup(fea_hbm, stars_hbm, postal_hbm, out_hbm,
                fea_v, idx_v, buf_v, table_sh, sem0, sem1, sem2, sem3):
    cid = lax.axis_index("c")
    sid = lax.axis_index("s")
    wid = sid * _NC + cid

    @pl.when(sid < 5)
    def _():
        pltpu.sync_copy(postal_hbm.at[pl.ds(sid * 200, 200)],
                        table_sh.at[pl.ds(sid * 200, 200)])
    @pl.when(sid == 5)
    def _():
        pltpu.sync_copy(stars_hbm, table_sh.at[pl.ds(_NUM_POSTAL, _NUM_STARS)])
    plsc.subcore_barrier()

    pltpu.sync_copy(fea_hbm.at[pl.ds(wid * _ITEMS_PER_W * 3, _ITEMS_PER_W * 3)],
                    fea_v)

    lane = lax.iota(jnp.int32, _L)

    @pl.loop(0, _ITEMS_PER_W // _L)
    def _(i):
        r = i * _L + lane
        stars = plsc.load_gather(fea_v, [3 * r + 1])
        postal = plsc.load_gather(fea_v, [3 * r + 2])
        plsc.store_scatter(idx_v, [2 * r], stars + _NUM_POSTAL)
        plsc.store_scatter(idx_v, [2 * r + 1], postal)

    base_item = wid * _ITEMS_PER_W
    sems = (sem0, sem1, sem2, sem3)
    gh = {}
    sh = {}

    def start_gather(c):
        b = c % _NBUF
        return pltpu.async_copy(
            table_sh.at[idx_v.at[pl.ds(c * _CHUNK, _CHUNK)]], buf_v.at[b], sems[b])

    def start_store(c):
        b = c % _NBUF
        return pltpu.async_copy(
            buf_v.at[b].reshape(_CHUNK // 2, 2 * _D),
            out_hbm.at[pl.ds(base_item + c * (_CHUNK // 2), _CHUNK // 2)],
            sems[b])

    for c in range(_NBUF):
        gh[c] = start_gather(c)
    for c in range(_N_CHUNKS):
        gh[c].wait()
        sh[c] = start_store(c)
        if c + _NBUF < _N_CHUNKS:
            sh[c].wait()
            gh[c + _NBUF] = start_gather(c + _NBUF)
    for c in range(_N_CHUNKS - _NBUF, _N_CHUNKS):
        sh[c].wait()


def kernel(item_fea, W_stars, W_postalcode):
    fea = item_fea.astype(jnp.int32).reshape(_B * 3)
    return _emb_lookup(fea, W_stars, W_postalcode)


# speedup vs baseline: 1.3271x; 1.3271x over previous
import dataclasses
import functools

import jax
import jax.numpy as jnp
from jax import lax
from jax.experimental import pallas as pl
from jax.experimental.pallas import tpu as pltpu
from jax.experimental.pallas import tpu_sc as plsc

_NUM_STARS = 10
_NUM_POSTAL = 1000
_D = 128
_B = 16384

_NC = 2
_NS = 16
_L = 16
_NW = _NC * _NS
_ITEMS_PER_W = _B // _NW
_ROWS_PER_W = 2 * _ITEMS_PER_W
_CHUNK = 128
_N_CHUNKS = _ROWS_PER_W // _CHUNK
_NBUF = 4


_CP = pltpu.CompilerParams()
if "needs_layout_passes" in pltpu.CompilerParams.__dataclass_fields__:
    _CP = dataclasses.replace(_CP, needs_layout_passes=False)


@functools.partial(
    pl.kernel,
    mesh=plsc.VectorSubcoreMesh(core_axis_name="c", subcore_axis_name="s"),
    compiler_params=_CP,
    out_type=jax.ShapeDtypeStruct((_B, 2 * _D), jnp.float32),
    scratch_types=[
        pltpu.VMEM((_ITEMS_PER_W,), jnp.int32),
        pltpu.VMEM((_ITEMS_PER_W,), jnp.int32),
        pltpu.VMEM((_ROWS_PER_W,), jnp.int32),
        pltpu.VMEM((_NBUF, _CHUNK, _D), jnp.float32),
        pltpu.VMEM_SHARED((_NUM_STARS + _NUM_POSTAL, _D), jnp.float32),
        pltpu.SemaphoreType.DMA,
        pltpu.SemaphoreType.DMA,
        pltpu.SemaphoreType.DMA,
        pltpu.SemaphoreType.DMA,
    ],
)
def _emb_lookup(s_hbm, p_hbm, stars_hbm, postal_hbm, out_hbm,
                cs_v, cp_v, idx_v, buf_v, table_sh, sem0, sem1, sem2, sem3):
    cid = lax.axis_index("c")
    sid = lax.axis_index("s")
    wid = sid * _NC + cid

    @pl.when(sid < 5)
    def _():
        pltpu.sync_copy(postal_hbm.at[pl.ds(sid * 200, 200)],
                        table_sh.at[pl.ds(sid * 200, 200)])
    @pl.when(sid == 5)
    def _():
        pltpu.sync_copy(stars_hbm, table_sh.at[pl.ds(_NUM_POSTAL, _NUM_STARS)])
    plsc.subcore_barrier()

    base_item = wid * _ITEMS_PER_W
    pltpu.sync_copy(s_hbm.at[pl.ds(base_item, _ITEMS_PER_W)], cs_v)
    pltpu.sync_copy(p_hbm.at[pl.ds(base_item, _ITEMS_PER_W)], cp_v)

    lane = lax.iota(jnp.int32, _L)

    @pl.loop(0, _ITEMS_PER_W // _L)
    def _(i):
        r = i * _L + lane
        stars = cs_v[pl.ds(i * _L, _L)]
        postal = cp_v[pl.ds(i * _L, _L)]
        plsc.store_scatter(idx_v, [2 * r], stars + _NUM_POSTAL)
        plsc.store_scatter(idx_v, [2 * r + 1], postal)

    sems = (sem0, sem1, sem2, sem3)
    gh = {}
    sh = {}

    def start_gather(c):
        b = c % _NBUF
        return pltpu.async_copy(
            table_sh.at[idx_v.at[pl.ds(c * _CHUNK, _CHUNK)]], buf_v.at[b], sems[b])

    def start_store(c):
        b = c % _NBUF
        return pltpu.async_copy(
            buf_v.at[b].reshape(_CHUNK // 2, 2 * _D),
            out_hbm.at[pl.ds(base_item + c * (_CHUNK // 2), _CHUNK // 2)],
            sems[b])

    for c in range(_NBUF):
        gh[c] = start_gather(c)
    for c in range(_N_CHUNKS):
        gh[c].wait()
        sh[c] = start_store(c)
        if c + _NBUF < _N_CHUNKS:
            sh[c].wait()
            gh[c + _NBUF] = start_gather(c + _NBUF)
    for c in range(_N_CHUNKS - _NBUF, _N_CHUNKS):
        sh[c].wait()


def kernel(item_fea, W_stars, W_postalcode):
    fea = item_fea.astype(jnp.int32)
    s_idx = fea[:, 1]
    p_idx = fea[:, 2]
    return _emb_lookup(s_idx, p_idx, W_stars, W_postalcode)
